# baseline (speedup 1.0000x reference)
import functools

import jax
import jax.numpy as jnp
from jax import lax
from jax.experimental import pallas as pl
from jax.experimental.pallas import tpu as pltpu
from jax.experimental.pallas import tpu_sc as plsc

SIGMA = 1.0
LR = 0.1
NUM = 1024
N = 512
NC = 2
NS = 16
NW = NC * NS
RPT = NUM // NW
L = 16
NCH = N // L

_mesh = functools.partial(
    plsc.VectorSubcoreMesh,
    core_axis_name="c",
    subcore_axis_name="s",
    num_cores=NC,
    num_subcores=NS,
)


def _tid():
    return lax.axis_index("c") * NS + lax.axis_index("s")


def _lanesum(v):
    iot = lax.iota(jnp.int32, L)
    for k in (8, 4, 2, 1):
        v = v + v.at[iot ^ k].get(mode="promise_in_bounds")
    return v[0]


def _dists_body(x_hbm, w_hbm, mins_hbm, x_v, w_v, red_v):
    tid = _tid()
    base = tid * RPT
    pltpu.sync_copy(x_hbm, x_v)
    pltpu.sync_copy(w_hbm.at[pl.ds(base, RPT)], w_v)

    def row_step(r, carry):
        best, bidx = carry

        def chunk_step(j, acc):
            d = w_v[r, pl.ds(j * L, L)] - x_v[pl.ds(j * L, L)]
            return acc + d * d

        acc = lax.fori_loop(0, NCH, chunk_step, jnp.zeros((L,), jnp.float32))
        dist = _lanesum(acc)
        take = dist < best
        best = jnp.where(take, dist, best)
        bidx = jnp.where(take, (base + r).astype(jnp.float32), bidx)
        return best, bidx

    best, bidx = lax.fori_loop(
        0, RPT, row_step, (jnp.float32(jnp.inf), jnp.float32(0.0))
    )
    iot = lax.iota(jnp.int32, L)
    red_v[:] = jnp.where(iot == 0, best, jnp.where(iot == 1, bidx, 0.0))
    pltpu.sync_copy(red_v, mins_hbm.at[tid])


def _update_body(x_hbm, w_hbm, nhb_hbm, mins_hbm, out_hbm,
                 x_v, w_v, mins_v, nhb_v, latt_v):
    tid = _tid()
    base = tid * RPT
    pltpu.sync_copy(x_hbm, x_v)
    pltpu.sync_copy(mins_hbm, mins_v)
    pltpu.sync_copy(w_hbm.at[pl.ds(base, RPT)], w_v)

    def red_step(t, carry):
        best, bidx = carry
        row = mins_v[t]
        v = row[0]
        take = v < best
        best = jnp.where(take, v, best)
        bidx = jnp.where(take, row[1], bidx)
        return best, bidx

    _, bidx = lax.fori_loop(
        0, NW, red_step, (jnp.float32(jnp.inf), jnp.float32(0.0))
    )
    bmu = bidx.astype(jnp.int32)

    pltpu.sync_copy(nhb_hbm.at[bmu, pl.ds(base, RPT)], nhb_v)
    for k in range(RPT // L):
        latt_v[pl.ds(k * L, L)] = jnp.exp(nhb_v[pl.ds(k * L, L)] * -0.5)

    def row_step(r, _):
        lo = (r // L) * L
        lc = latt_v[pl.ds(lo, L)]
        s = lc.at[jnp.full((L,), r - lo, jnp.int32)].get(
            mode="promise_in_bounds") * LR

        def chunk_step(j, __):
            w = w_v[r, pl.ds(j * L, L)]
            w_v[r, pl.ds(j * L, L)] = w + s * (x_v[pl.ds(j * L, L)] - w)
            return 0

        return lax.fori_loop(0, NCH, chunk_step, 0)

    lax.fori_loop(0, RPT, row_step, 0)
    pltpu.sync_copy(w_v, out_hbm.at[pl.ds(base, RPT)])


_dists_call = pl.kernel(
    _dists_body,
    out_type=jax.ShapeDtypeStruct((NW, L), jnp.float32),
    mesh=_mesh(),
    scratch_types=[
        pltpu.VMEM((N,), jnp.float32),
        pltpu.VMEM((RPT, N), jnp.float32),
        pltpu.VMEM((L,), jnp.float32),
    ],
)

_update_call = pl.kernel(
    _update_body,
    out_type=jax.ShapeDtypeStruct((NUM, N), jnp.float32),
    mesh=_mesh(),
    scratch_types=[
        pltpu.VMEM((N,), jnp.float32),
        pltpu.VMEM((RPT, N), jnp.float32),
        pltpu.VMEM((NW, L), jnp.float32),
        pltpu.VMEM((RPT,), jnp.float32),
        pltpu.VMEM((RPT,), jnp.float32),
    ],
)


@jax.jit
def kernel(x, W, nhb_dists):
    xf = x.reshape(N)
    mins = _dists_call(xf, W)
    return _update_call(xf, W, nhb_dists, mins)


# speedup vs baseline: 1.2681x; 1.2681x over previous
import jax
import jax.numpy as jnp
from jax import lax
from jax.experimental import pallas as pl
from jax.experimental.pallas import tpu as pltpu
from jax.experimental.pallas import tpu_sc as plsc

SIGMA = 1.0
LR = 0.1
NUM = 1024
N = 512
NC = 2
NS = 16
NW = NC * NS
RPT = NUM // NW
L = 16
NCH = N // L
BLK = 128

_mesh = plsc.VectorSubcoreMesh(
    core_axis_name="c", subcore_axis_name="s", num_cores=NC, num_subcores=NS)


def _lanesum(v):
    iot = lax.iota(jnp.int32, L)
    for k in (8, 4, 2, 1):
        v = v + v.at[iot ^ k].get(mode="promise_in_bounds")
    return v[0]


def _dists_body(x_hbm, w_hbm, mins_hbm, x_v, w_v, red_v):
    tid = lax.axis_index("c") * NS + lax.axis_index("s")
    base = tid * RPT
    pltpu.sync_copy(x_hbm, x_v)
    pltpu.sync_copy(w_hbm.at[pl.ds(base, RPT)], w_v)
    xc = [x_v[pl.ds(j * L, L)] for j in range(NCH)]

    def row_step(r, carry):
        best, bidx = carry
        accs = [jnp.zeros((L,), jnp.float32) for _ in range(4)]
        for j in range(NCH):
            d = w_v[r, pl.ds(j * L, L)] - xc[j]
            accs[j % 4] = accs[j % 4] + d * d
        dist = _lanesum((accs[0] + accs[1]) + (accs[2] + accs[3]))
        take = dist < best
        best = jnp.where(take, dist, best)
        bidx = jnp.where(take, (base + r).astype(jnp.float32), bidx)
        return best, bidx

    best, bidx = lax.fori_loop(
        0, RPT, row_step, (jnp.float32(jnp.inf), jnp.float32(0.0))
    )
    iot = lax.iota(jnp.int32, L)
    red_v[:] = jnp.where(iot == 0, best, jnp.where(iot == 1, bidx, 0.0))
    pltpu.sync_copy(red_v, mins_hbm.at[tid])


_dists_call = pl.kernel(
    _dists_body,
    out_type=jax.ShapeDtypeStruct((NW, L), jnp.float32),
    mesh=_mesh,
    scratch_types=[
        pltpu.VMEM((N,), jnp.float32),
        pltpu.VMEM((RPT, N), jnp.float32),
        pltpu.VMEM((L,), jnp.float32),
    ],
)


def _bmu_body(mins_ref, nhb_hbm, latt_ref, sem):
    vals = mins_ref[:, 0:1]
    idxs = mins_ref[:, 1:2]
    gmin = jnp.min(vals)
    bidx = jnp.min(jnp.where(vals == gmin, idxs, jnp.float32(2e9)))
    bmu = bidx.astype(jnp.int32)
    copy = pltpu.make_async_copy(nhb_hbm.at[bmu], latt_ref, sem)
    copy.start()
    copy.wait()
    latt_ref[...] = jnp.exp(latt_ref[...] * -0.5)


_bmu_call = pl.pallas_call(
    _bmu_body,
    out_shape=jax.ShapeDtypeStruct((NUM,), jnp.float32),
    in_specs=[
        pl.BlockSpec(memory_space=pltpu.VMEM),
        pl.BlockSpec(memory_space=pl.ANY),
    ],
    out_specs=pl.BlockSpec(memory_space=pltpu.VMEM),
    scratch_shapes=[pltpu.SemaphoreType.DMA],
)


def _upd_body(w_ref, latt_ref, x_ref, out_ref):
    w = w_ref[...]
    out_ref[...] = w + (LR * latt_ref[...]) * (x_ref[...] - w)


_upd_call = pl.pallas_call(
    _upd_body,
    grid=(NUM // BLK,),
    in_specs=[
        pl.BlockSpec((BLK, N), lambda i: (i, 0)),
        pl.BlockSpec((BLK, 1), lambda i: (i, 0)),
        pl.BlockSpec((1, N), lambda i: (0, 0)),
    ],
    out_specs=pl.BlockSpec((BLK, N), lambda i: (i, 0)),
    out_shape=jax.ShapeDtypeStruct((NUM, N), jnp.float32),
)


@jax.jit
def kernel(x, W, nhb_dists):
    mins = _dists_call(x.reshape(N), W)
    latt = _bmu_call(mins, nhb_dists)
    return _upd_call(W, latt.reshape(NUM, 1), x.reshape(1, N))


# speedup vs baseline: 1.3872x; 1.0939x over previous
import jax
import jax.numpy as jnp
from jax import lax
from jax.experimental import pallas as pl
from jax.experimental.pallas import tpu as pltpu
from jax.experimental.pallas import tpu_sc as plsc

SIGMA = 1.0
LR = 0.1
NUM = 1024
N = 512
NC = 2
NS = 16
NW = NC * NS
RPT = NUM // NW
HRPT = RPT // 2
L = 16
NCH = N // L
BLK = 128

_mesh = plsc.VectorSubcoreMesh(
    core_axis_name="c", subcore_axis_name="s", num_cores=NC, num_subcores=NS)


def _lanesum(v):
    iot = lax.iota(jnp.int32, L)
    for k in (8, 4, 2, 1):
        v = v + v.at[iot ^ k].get(mode="promise_in_bounds")
    return v[0]


def _dists_body(x_hbm, w_hbm, mins_hbm, x_v, w_v, red_v, sem0, sem1):
    tid = lax.axis_index("c") * NS + lax.axis_index("s")
    base = tid * RPT
    cp0 = pltpu.async_copy(
        w_hbm.at[pl.ds(base, HRPT)], w_v.at[pl.ds(0, HRPT)], sem0)
    cp1 = pltpu.async_copy(
        w_hbm.at[pl.ds(base + HRPT, HRPT)], w_v.at[pl.ds(HRPT, HRPT)], sem1)
    pltpu.sync_copy(x_hbm, x_v)
    xc = [x_v[pl.ds(j * L, L)] for j in range(NCH)]

    def row_step(r, carry):
        best, bidx = carry
        accs = [jnp.zeros((L,), jnp.float32) for _ in range(4)]
        for j in range(NCH):
            d = w_v[r, pl.ds(j * L, L)] - xc[j]
            accs[j % 4] = accs[j % 4] + d * d
        dist = _lanesum((accs[0] + accs[1]) + (accs[2] + accs[3]))
        take = dist < best
        best = jnp.where(take, dist, best)
        bidx = jnp.where(take, (base + r).astype(jnp.float32), bidx)
        return best, bidx

    cp0.wait()
    carry = lax.fori_loop(
        0, HRPT, row_step, (jnp.float32(jnp.inf), jnp.float32(0.0))
    )
    cp1.wait()
    best, bidx = lax.fori_loop(HRPT, RPT, row_step, carry)
    iot = lax.iota(jnp.int32, L)
    red_v[:] = jnp.where(iot == 0, best, jnp.where(iot == 1, bidx, 0.0))
    pltpu.sync_copy(red_v, mins_hbm.at[tid])


_dists_call = pl.kernel(
    _dists_body,
    out_type=jax.ShapeDtypeStruct((NW, L), jnp.float32),
    mesh=_mesh,
    scratch_types=[
        pltpu.VMEM((N,), jnp.float32),
        pltpu.VMEM((RPT, N), jnp.float32),
        pltpu.VMEM((L,), jnp.float32),
        pltpu.SemaphoreType.DMA,
        pltpu.SemaphoreType.DMA,
    ],
)


def _upd_body(mins_ref, nhb_hbm, x_ref, w_ref, out_ref,
              latt_row, latt_col, sem):
    i = pl.program_id(0)

    @pl.when(i == 0)
    def _():
        vals = mins_ref[:, 0:1]
        idxs = mins_ref[:, 1:2]
        gmin = jnp.min(vals)
        bidx = jnp.min(jnp.where(vals == gmin, idxs, jnp.float32(2e9)))
        bmu = bidx.astype(jnp.int32)
        cp = pltpu.make_async_copy(nhb_hbm.at[bmu], latt_row, sem)
        cp.start()
        cp.wait()
        lr = jnp.exp(latt_row[...] * -0.5)
        eye = (lax.broadcasted_iota(jnp.int32, (BLK, BLK), 0) ==
               lax.broadcasted_iota(jnp.int32, (BLK, BLK), 1)
               ).astype(jnp.float32)
        for k in range(NUM // BLK):
            seg = lr[k * BLK:(k + 1) * BLK].reshape(1, BLK)
            latt_col[pl.ds(k * BLK, BLK), :] = lax.dot_general(
                eye, seg, (((1,), (1,)), ((), ())),
                precision=lax.Precision.HIGHEST)

    w = w_ref[...]
    lc = latt_col[pl.ds(i * BLK, BLK), :]
    out_ref[...] = w + (LR * lc) * (x_ref[...] - w)


_upd_call = pl.pallas_call(
    _upd_body,
    grid=(NUM // BLK,),
    in_specs=[
        pl.BlockSpec((NW, L), lambda i: (0, 0)),
        pl.BlockSpec(memory_space=pl.ANY),
        pl.BlockSpec((1, N), lambda i: (0, 0)),
        pl.BlockSpec((BLK, N), lambda i: (i, 0)),
    ],
    out_specs=pl.BlockSpec((BLK, N), lambda i: (i, 0)),
    out_shape=jax.ShapeDtypeStruct((NUM, N), jnp.float32),
    scratch_shapes=[
        pltpu.VMEM((NUM,), jnp.float32),
        pltpu.VMEM((NUM, 1), jnp.float32),
        pltpu.SemaphoreType.DMA,
    ],
)


@jax.jit
def kernel(x, W, nhb_dists):
    mins = _dists_call(x.reshape(N), W)
    return _upd_call(mins, nhb_dists, x.reshape(1, N), W)


# speedup vs baseline: 1.4169x; 1.0215x over previous
import jax
import jax.numpy as jnp
from jax import lax
from jax.experimental import pallas as pl
from jax.experimental.pallas import tpu as pltpu
from jax.experimental.pallas import tpu_sc as plsc

SIGMA = 1.0
LR = 0.1
NUM = 1024
N = 512
NC = 2
NS = 16
NW = NC * NS
RPT = NUM // NW
HRPT = RPT // 2
L = 16
NCH = N // L
BLK = 128

_mesh = plsc.VectorSubcoreMesh(
    core_axis_name="c", subcore_axis_name="s", num_cores=NC, num_subcores=NS)


def _lanemin(v):
    iot = lax.iota(jnp.int32, L)
    for k in (8, 4, 2, 1):
        v = jnp.minimum(v, v.at[iot ^ k].get(mode="promise_in_bounds"))
    return v


def _dists_body(x_hbm, w_hbm, mins_hbm, x_v, w_v, red_v, acc_v, sem0, sem1):
    tid = lax.axis_index("c") * NS + lax.axis_index("s")
    base = tid * RPT
    cp0 = pltpu.async_copy(
        w_hbm.at[pl.ds(base, HRPT)], w_v.at[pl.ds(0, HRPT)], sem0)
    cp1 = pltpu.async_copy(
        w_hbm.at[pl.ds(base + HRPT, HRPT)], w_v.at[pl.ds(HRPT, HRPT)], sem1)
    pltpu.sync_copy(x_hbm, x_v)
    xc = [x_v[pl.ds(j * L, L)] for j in range(NCH)]
    iot = lax.iota(jnp.int32, L)

    def row_step(i, gbase):
        accs = [jnp.zeros((L,), jnp.float32) for _ in range(4)]
        for j in range(NCH):
            d = w_v[gbase + i, pl.ds(j * L, L)] - xc[j]
            accs[j % 4] = accs[j % 4] + d * d
        acc_v[pl.ds(i * L, L)] = (accs[0] + accs[1]) + (accs[2] + accs[3])
        return gbase

    bestvec = jnp.full((L,), jnp.inf, jnp.float32)
    bestrow = jnp.zeros((L,), jnp.float32)
    for g, cp in ((0, cp0), (1, cp1)):
        cp.wait()
        lax.fori_loop(0, L, row_step, g * L)
        cols = [plsc.load_gather(acc_v, [iot * L + j])
                for j in range(L)]
        for step in (8, 4, 2, 1):
            cols = [cols[t] + cols[t + step] for t in range(step)]
        dists16 = cols[0]
        take = dists16 < bestvec
        bestvec = jnp.where(take, dists16, bestvec)
        bestrow = jnp.where(take, (base + g * L + iot).astype(jnp.float32),
                            bestrow)

    gminv = _lanemin(bestvec)
    cand = jnp.where(bestvec == gminv, bestrow, jnp.float32(2e9))
    gidxv = _lanemin(cand)
    red_v[:] = jnp.where(iot == 0, gminv, jnp.where(iot == 1, gidxv, 0.0))
    pltpu.sync_copy(red_v, mins_hbm.at[tid])


_dists_call = pl.kernel(
    _dists_body,
    out_type=jax.ShapeDtypeStruct((NW, L), jnp.float32),
    mesh=_mesh,
    compiler_params=pltpu.CompilerParams(needs_layout_passes=False),
    scratch_types=[
        pltpu.VMEM((N,), jnp.float32),
        pltpu.VMEM((RPT, N), jnp.float32),
        pltpu.VMEM((L,), jnp.float32),
        pltpu.VMEM((L * L,), jnp.float32),
        pltpu.SemaphoreType.DMA,
        pltpu.SemaphoreType.DMA,
    ],
)


def _upd_body(mins_ref, nhb_hbm, x_ref, w_ref, out_ref,
              latt_row, latt_col, sem):
    i = pl.program_id(0)

    @pl.when(i == 0)
    def _():
        vals = mins_ref[:, 0:1]
        idxs = mins_ref[:, 1:2]
        gmin = jnp.min(vals)
        bidx = jnp.min(jnp.where(vals == gmin, idxs, jnp.float32(2e9)))
        bmu = bidx.astype(jnp.int32)
        cp = pltpu.make_async_copy(nhb_hbm.at[bmu], latt_row, sem)
        cp.start()
        cp.wait()
        lr = jnp.exp(latt_row[...] * -0.5)
        eye = (lax.broadcasted_iota(jnp.int32, (BLK, BLK), 0) ==
               lax.broadcasted_iota(jnp.int32, (BLK, BLK), 1)
               ).astype(jnp.float32)
        for k in range(NUM // BLK):
            seg = lr[k * BLK:(k + 1) * BLK].reshape(1, BLK)
            latt_col[pl.ds(k * BLK, BLK), :] = lax.dot_general(
                eye, seg, (((1,), (1,)), ((), ())),
                precision=lax.Precision.HIGHEST)

    w = w_ref[...]
    lc = latt_col[pl.ds(i * BLK, BLK), :]
    out_ref[...] = w + (LR * lc) * (x_ref[...] - w)


_upd_call = pl.pallas_call(
    _upd_body,
    grid=(NUM // BLK,),
    in_specs=[
        pl.BlockSpec((NW, L), lambda i: (0, 0)),
        pl.BlockSpec(memory_space=pl.ANY),
        pl.BlockSpec((1, N), lambda i: (0, 0)),
        pl.BlockSpec((BLK, N), lambda i: (i, 0)),
    ],
    out_specs=pl.BlockSpec((BLK, N), lambda i: (i, 0)),
    out_shape=jax.ShapeDtypeStruct((NUM, N), jnp.float32),
    scratch_shapes=[
        pltpu.VMEM((NUM,), jnp.float32),
        pltpu.VMEM((NUM, 1), jnp.float32),
        pltpu.SemaphoreType.DMA,
    ],
)


@jax.jit
def kernel(x, W, nhb_dists):
    mins = _dists_call(x.reshape(N), W)
    return _upd_call(mins, nhb_dists, x.reshape(1, N), W)


# speedup vs baseline: 1.4567x; 1.0281x over previous
import jax
import jax.numpy as jnp
from jax import lax
from jax.experimental import pallas as pl
from jax.experimental.pallas import tpu as pltpu
from jax.experimental.pallas import tpu_sc as plsc

SIGMA = 1.0
LR = 0.1
NUM = 1024
N = 512
NC = 2
NS = 16
NW = NC * NS
NSC = NUM // 2
RPT = NSC // NW
L = 16
NCH = N // L
BLK = 128

_mesh = plsc.VectorSubcoreMesh(
    core_axis_name="c", subcore_axis_name="s", num_cores=NC, num_subcores=NS)


def _lanemin(v):
    iot = lax.iota(jnp.int32, L)
    for k in (8, 4, 2, 1):
        v = jnp.minimum(v, v.at[iot ^ k].get(mode="promise_in_bounds"))
    return v


def _dists_body(x_hbm, w_hbm, mins_hbm, x_v, w_v, red_v, acc_v, sem0):
    tid = lax.axis_index("c") * NS + lax.axis_index("s")
    base = tid * RPT
    cp0 = pltpu.async_copy(w_hbm.at[pl.ds(base, RPT)], w_v, sem0)
    pltpu.sync_copy(x_hbm, x_v)
    xc = [x_v[pl.ds(j * L, L)] for j in range(NCH)]
    iot = lax.iota(jnp.int32, L)

    def row_step(i, _):
        accs = [jnp.zeros((L,), jnp.float32) for _ in range(4)]
        for j in range(NCH):
            d = w_v[i, pl.ds(j * L, L)] - xc[j]
            accs[j % 4] = accs[j % 4] + d * d
        acc_v[pl.ds(i * L, L)] = (accs[0] + accs[1]) + (accs[2] + accs[3])
        return 0

    cp0.wait()
    lax.fori_loop(0, RPT, row_step, 0)
    cols = [plsc.load_gather(acc_v, [iot * L + j]) for j in range(L)]
    for step in (8, 4, 2, 1):
        cols = [cols[t] + cols[t + step] for t in range(step)]
    dists16 = cols[0]

    gminv = _lanemin(dists16)
    cand = jnp.where(dists16 == gminv,
                     (base + iot).astype(jnp.float32), jnp.float32(2e9))
    gidxv = _lanemin(cand)
    red_v[:] = jnp.where(iot == 0, gminv, jnp.where(iot == 1, gidxv, 0.0))
    pltpu.sync_copy(red_v, mins_hbm.at[tid])


_dists_call = pl.kernel(
    _dists_body,
    out_type=jax.ShapeDtypeStruct((NW, L), jnp.float32),
    mesh=_mesh,
    compiler_params=pltpu.CompilerParams(needs_layout_passes=False),
    scratch_types=[
        pltpu.VMEM((N,), jnp.float32),
        pltpu.VMEM((RPT, N), jnp.float32),
        pltpu.VMEM((L,), jnp.float32),
        pltpu.VMEM((L * L,), jnp.float32),
        pltpu.SemaphoreType.DMA,
    ],
)


def _tcdists_body(x_ref, w_ref, out_ref):
    d = w_ref[...] - x_ref[...]
    out_ref[...] = jnp.sum(d * d, axis=1, keepdims=True)


_tcdists_call = pl.pallas_call(
    _tcdists_body,
    grid=(NSC // BLK,),
    in_specs=[
        pl.BlockSpec((1, N), lambda i: (0, 0)),
        pl.BlockSpec((BLK, N), lambda i: (i + NSC // BLK, 0)),
    ],
    out_specs=pl.BlockSpec((BLK, 1), lambda i: (i, 0)),
    out_shape=jax.ShapeDtypeStruct((NSC, 1), jnp.float32),
)


def _upd_body(mins_ref, d2_ref, nhb_hbm, x_ref, w_ref, out_ref,
              latt_row, latt_col, sem):
    i = pl.program_id(0)

    @pl.when(i == 0)
    def _():
        vals = mins_ref[:, 0:1]
        idxs = mins_ref[:, 1:2]
        m1 = jnp.min(vals)
        i1 = jnp.min(jnp.where(vals == m1, idxs, jnp.float32(2e9)))
        v2 = d2_ref[...]
        m2 = jnp.min(v2)
        rows2 = lax.broadcasted_iota(jnp.int32, (NSC, 1), 0).astype(jnp.float32) + NSC
        i2 = jnp.min(jnp.where(v2 == m2, rows2, jnp.float32(2e9)))
        bidx = jnp.where(m1 <= m2, i1, i2)
        bmu = bidx.astype(jnp.int32)
        cp = pltpu.make_async_copy(nhb_hbm.at[bmu], latt_row, sem)
        cp.start()
        cp.wait()
        lr = jnp.exp(latt_row[...] * -0.5)
        eye = (lax.broadcasted_iota(jnp.int32, (BLK, BLK), 0) ==
               lax.broadcasted_iota(jnp.int32, (BLK, BLK), 1)
               ).astype(jnp.float32)
        for k in range(NUM // BLK):
            seg = lr[k * BLK:(k + 1) * BLK].reshape(1, BLK)
            latt_col[pl.ds(k * BLK, BLK), :] = lax.dot_general(
                eye, seg, (((1,), (1,)), ((), ())),
                precision=lax.Precision.HIGHEST)

    w = w_ref[...]
    lc = latt_col[pl.ds(i * BLK, BLK), :]
    out_ref[...] = w + (LR * lc) * (x_ref[...] - w)


_upd_call = pl.pallas_call(
    _upd_body,
    grid=(NUM // BLK,),
    in_specs=[
        pl.BlockSpec((NW, L), lambda i: (0, 0)),
        pl.BlockSpec((NSC, 1), lambda i: (0, 0)),
        pl.BlockSpec(memory_space=pl.ANY),
        pl.BlockSpec((1, N), lambda i: (0, 0)),
        pl.BlockSpec((BLK, N), lambda i: (i, 0)),
    ],
    out_specs=pl.BlockSpec((BLK, N), lambda i: (i, 0)),
    out_shape=jax.ShapeDtypeStruct((NUM, N), jnp.float32),
    scratch_shapes=[
        pltpu.VMEM((NUM,), jnp.float32),
        pltpu.VMEM((NUM, 1), jnp.float32),
        pltpu.SemaphoreType.DMA,
    ],
)


@jax.jit
def kernel(x, W, nhb_dists):
    x2 = x.reshape(1, N)
    mins = _dists_call(x.reshape(N), W)
    d2 = _tcdists_call(x2, W)
    return _upd_call(mins, d2, nhb_dists, x2, W)


# speedup vs baseline: 1.4801x; 1.0161x over previous
import jax
import jax.numpy as jnp
from jax import lax
from jax.experimental import pallas as pl
from jax.experimental.pallas import tpu as pltpu
from jax.experimental.pallas import tpu_sc as plsc

SIGMA = 1.0
LR = 0.1
NUM = 1024
N = 512
NC = 2
NS = 16
NW = NC * NS
NSC = NUM // 4
NTC = NUM - NSC
RPT = NSC // NW
L = 16
NCH = N // L
BLK = 128

_mesh = plsc.VectorSubcoreMesh(
    core_axis_name="c", subcore_axis_name="s", num_cores=NC, num_subcores=NS)


def _lanemin(v):
    iot = lax.iota(jnp.int32, L)
    for k in (8, 4, 2, 1):
        v = jnp.minimum(v, v.at[iot ^ k].get(mode="promise_in_bounds"))
    return v


def _dists_body(x_hbm, w_hbm, mins_hbm, x_v, w_v, red_v, acc_v, sem0):
    tid = lax.axis_index("c") * NS + lax.axis_index("s")
    base = tid * RPT
    cp0 = pltpu.async_copy(w_hbm.at[pl.ds(base, RPT)], w_v, sem0)
    pltpu.sync_copy(x_hbm, x_v)
    xc = [x_v[pl.ds(j * L, L)] for j in range(NCH)]
    iot = lax.iota(jnp.int32, L)

    def row_step(i, _):
        accs = [jnp.zeros((L,), jnp.float32) for _ in range(4)]
        for j in range(NCH):
            d = w_v[i, pl.ds(j * L, L)] - xc[j]
            accs[j % 4] = accs[j % 4] + d * d
        acc_v[pl.ds(i * L, L)] = (accs[0] + accs[1]) + (accs[2] + accs[3])
        return 0

    cp0.wait()
    lax.fori_loop(0, RPT, row_step, 0)
    cols = [plsc.load_gather(acc_v, [iot * L + j]) for j in range(L)]
    for step in (8, 4, 2, 1):
        cols = [cols[t] + cols[t + step] for t in range(step)]
    dists16 = cols[0]
    if RPT < L:
        dists16 = jnp.where(iot < RPT, dists16, jnp.float32(jnp.inf))

    gminv = _lanemin(dists16)
    cand = jnp.where(dists16 == gminv,
                     (base + iot).astype(jnp.float32), jnp.float32(2e9))
    gidxv = _lanemin(cand)
    red_v[:] = jnp.where(iot == 0, gminv, jnp.where(iot == 1, gidxv, 0.0))
    pltpu.sync_copy(red_v, mins_hbm.at[tid])


_dists_call = pl.kernel(
    _dists_body,
    out_type=jax.ShapeDtypeStruct((NW, L), jnp.float32),
    mesh=_mesh,
    compiler_params=pltpu.CompilerParams(needs_layout_passes=False),
    scratch_types=[
        pltpu.VMEM((N,), jnp.float32),
        pltpu.VMEM((RPT, N), jnp.float32),
        pltpu.VMEM((L,), jnp.float32),
        pltpu.VMEM((L * L,), jnp.float32),
        pltpu.SemaphoreType.DMA,
    ],
)


def _tcdists_body(x_ref, w_ref, out_ref):
    d = w_ref[...] - x_ref[...]
    out_ref[...] = jnp.sum(d * d, axis=1, keepdims=True)


_tcdists_call = pl.pallas_call(
    _tcdists_body,
    grid=(NTC // BLK,),
    in_specs=[
        pl.BlockSpec((1, N), lambda i: (0, 0)),
        pl.BlockSpec((BLK, N), lambda i: (i + NSC // BLK, 0)),
    ],
    out_specs=pl.BlockSpec((BLK, 1), lambda i: (i, 0)),
    out_shape=jax.ShapeDtypeStruct((NTC, 1), jnp.float32),
)


def _upd_body(mins_ref, d2_ref, nhb_hbm, x_ref, w_ref, out_ref,
              latt_row, latt_col, sem):
    i = pl.program_id(0)

    @pl.when(i == 0)
    def _():
        vals = mins_ref[:, 0:1]
        idxs = mins_ref[:, 1:2]
        m1 = jnp.min(vals)
        i1 = jnp.min(jnp.where(vals == m1, idxs, jnp.float32(2e9)))
        v2 = d2_ref[...]
        m2 = jnp.min(v2)
        rows2 = lax.broadcasted_iota(jnp.int32, (NTC, 1), 0).astype(jnp.float32) + NSC
        i2 = jnp.min(jnp.where(v2 == m2, rows2, jnp.float32(2e9)))
        bidx = jnp.where(m1 <= m2, i1, i2)
        bmu = bidx.astype(jnp.int32)
        cp = pltpu.make_async_copy(nhb_hbm.at[bmu], latt_row, sem)
        cp.start()
        cp.wait()
        lr = jnp.exp(latt_row[...] * -0.5)
        eye = (lax.broadcasted_iota(jnp.int32, (BLK, BLK), 0) ==
               lax.broadcasted_iota(jnp.int32, (BLK, BLK), 1)
               ).astype(jnp.float32)
        for k in range(NUM // BLK):
            seg = lr[k * BLK:(k + 1) * BLK].reshape(1, BLK)
            latt_col[pl.ds(k * BLK, BLK), :] = lax.dot_general(
                eye, seg, (((1,), (1,)), ((), ())),
                precision=lax.Precision.HIGHEST)

    w = w_ref[...]
    lc = latt_col[pl.ds(i * BLK, BLK), :]
    out_ref[...] = w + (LR * lc) * (x_ref[...] - w)


_upd_call = pl.pallas_call(
    _upd_body,
    grid=(NUM // BLK,),
    in_specs=[
        pl.BlockSpec((NW, L), lambda i: (0, 0)),
        pl.BlockSpec((NTC, 1), lambda i: (0, 0)),
        pl.BlockSpec(memory_space=pl.ANY),
        pl.BlockSpec((1, N), lambda i: (0, 0)),
        pl.BlockSpec((BLK, N), lambda i: (i, 0)),
    ],
    out_specs=pl.BlockSpec((BLK, N), lambda i: (i, 0)),
    out_shape=jax.ShapeDtypeStruct((NUM, N), jnp.float32),
    scratch_shapes=[
        pltpu.VMEM((NUM,), jnp.float32),
        pltpu.VMEM((NUM, 1), jnp.float32),
        pltpu.SemaphoreType.DMA,
    ],
)


@jax.jit
def kernel(x, W, nhb_dists):
    x2 = x.reshape(1, N)
    mins = _dists_call(x.reshape(N), W)
    d2 = _tcdists_call(x2, W)
    return _upd_call(mins, d2, nhb_dists, x2, W)
